# baseline (speedup 1.0000x reference)
import jax
import jax.numpy as jnp
from jax import lax
from jax.experimental import pallas as pl
from jax.experimental.pallas import tpu as pltpu
from jax.experimental.pallas import tpu_sc as plsc

N = 10000
E = 100000
D = 128
H = 64
CH = 256
NSUB = 16
CPT = 26
EP = CH * CPT * NSUB
NROW = EP // CH
NPAD = 10240
TR = NPAD // NSUB
FIN = 128
NF = TR // FIN
CR = 3 * NPAD // NSUB

_MESH = plsc.VectorSubcoreMesh(core_axis_name="c", subcore_axis_name="s")
_SC_PARAMS = pltpu.CompilerParams(use_tc_tiling_on_sc=False, needs_layout_passes=False)


def _count_body(d2_f, d2_c, d2_cb, recip, acc2d, dbuf, cloc, rbuf, tbuf):
    c = lax.axis_index("c")
    s = lax.axis_index("s")

    def zloc(m, carry):
        cloc[pl.ds(m * 16, 16)] = jnp.zeros((16,), jnp.float32)
        return carry

    lax.fori_loop(0, 3 * NPAD // 16, zloc, 0)

    @pl.when(c == 0)
    def _():
        ones16 = jnp.full((16,), 1.0, jnp.float32)
        for e, d2 in enumerate((d2_f, d2_c, d2_cb)):
            pltpu.sync_copy(d2.at[pl.ds(s * CPT, CPT)], dbuf)
            off = e * NPAD

            def hist(r, carry):
                for j in range(CH // 16):
                    v = dbuf[r, pl.ds(j * 16, 16)] + off
                    plsc.addupdate_scatter(cloc, [v], ones16)
                return carry

            lax.fori_loop(0, CPT, hist, 0)
        pltpu.sync_copy(cloc, acc2d.at[s])

    plsc.subcore_barrier()

    @pl.when(c == 0)
    def _():
        def zbuf(m, carry):
            rbuf[pl.ds(m * 16, 16)] = jnp.zeros((16,), jnp.float32)
            return carry

        lax.fori_loop(0, CR // 16, zbuf, 0)
        for t in range(NSUB):
            pltpu.sync_copy(acc2d.at[t, pl.ds(s * CR, CR)], tbuf)

            def accum(m, carry):
                rbuf[pl.ds(m * 16, 16)] = (rbuf[pl.ds(m * 16, 16)]
                                           + tbuf[pl.ds(m * 16, 16)])
                return carry

            lax.fori_loop(0, CR // 16, accum, 0)

        def rec(m, carry):
            v = rbuf[pl.ds(m * 16, 16)]
            rbuf[pl.ds(m * 16, 16)] = 1.0 / jnp.maximum(v, 1.0)
            return carry

        lax.fori_loop(0, CR // 16, rec, 0)
        pltpu.sync_copy(rbuf, recip.at[pl.ds(s * CR, CR)])


_count_kernel = pl.kernel(
    _count_body,
    out_type=jax.ShapeDtypeStruct((3 * NPAD,), jnp.float32),
    mesh=_MESH,
    compiler_params=_SC_PARAMS,
    scratch_types=[
        pltpu.VMEM_SHARED((NSUB, 3 * NPAD), jnp.float32),
        pltpu.VMEM((CPT, CH), jnp.int32),
        pltpu.VMEM((3 * NPAD,), jnp.float32),
        pltpu.VMEM((CR,), jnp.float32),
        pltpu.VMEM((CR,), jnp.float32),
    ],
)


def _make_layer_kernel(relu: bool, tbn: int):
    def body(s2f, d2f, s2c, d2c, s2cb, d2cb,
             tabF, tabC, tabCB, recip,
             hu, hi_,
             accA, accB,
             sbuf, dbuf, rows, rAf, rBf, semA, semB):
        c = lax.axis_index("c")
        s = lax.axis_index("s")
        rs = s * TR
        half = c * tbn

        bufA = rows.at[0, pl.ds(0, FIN)]
        bufB = rows.at[1, pl.ds(0, FIN)]

        def fill_zero(r, carry):
            for j in range(4):
                rows[0, r, pl.ds(j * 16, 16)] = jnp.zeros((16,), jnp.float32)
            return carry

        def zero_acc(acc):
            for k in range(NF):
                pltpu.sync_copy(bufA, acc.at[pl.ds(rs + k * FIN, FIN)])

        lax.fori_loop(0, FIN, fill_zero, 0)
        zero_acc(accA)
        zero_acc(accB)
        plsc.subcore_barrier()

        def do_etype(s2, d2, table, acc):
            pltpu.sync_copy(s2.at[pl.ds(s * CPT, CPT)], sbuf)
            pltpu.sync_copy(d2.at[pl.ds(s * CPT, CPT)], dbuf)

            def addoff(r, carry):
                for j in range(CH // 16):
                    sbuf[r, pl.ds(j * 16, 16)] = sbuf[r, pl.ds(j * 16, 16)] + half
                return carry

            lax.fori_loop(0, CPT, addoff, 0)

            pltpu.async_copy(table.at[sbuf.at[0]], rows.at[0], semA)
            pltpu.async_copy(table.at[sbuf.at[1]], rows.at[1], semB)

            def grp(g, carry):
                for k, sem in ((0, semA), (1, semB)):
                    ch = g * 2 + k
                    pltpu.make_async_copy(
                        table.at[sbuf.at[0]], rows.at[k], sem).wait()
                    pltpu.sync_copy(rows.at[k], acc.at[dbuf.at[ch]], add=True)
                    nxt = ch + 2

                    @pl.when(nxt < CPT)
                    def _():
                        pltpu.async_copy(
                            table.at[sbuf.at[nxt]], rows.at[k], sem)
                return carry

            lax.fori_loop(0, CPT // 2, grp, 0)

        do_etype(s2f, d2f, tabF, accA)
        do_etype(s2cb, d2cb, tabCB, accB)
        plsc.subcore_barrier()

        for k in range(NF):
            ro = rs + k * FIN
            pltpu.sync_copy(accA.at[pl.ds(ro, FIN)], bufA)
            pltpu.sync_copy(accB.at[pl.ds(ro, FIN)], bufB)
            pltpu.sync_copy(recip.at[pl.ds(ro, FIN)], rAf)
            pltpu.sync_copy(recip.at[pl.ds(2 * NPAD + ro, FIN)], rBf)

            def fin_u(r, carry):
                ri = jnp.full((16,), r, jnp.int32)
                ra = plsc.load_gather(rAf, [ri])
                rb = plsc.load_gather(rBf, [ri])
                for j in range(4):
                    a = rows[0, r, pl.ds(j * 16, 16)]
                    b = rows[1, r, pl.ds(j * 16, 16)]
                    h = 0.5 * (a * ra + b * rb)
                    if relu:
                        h = jnp.maximum(h, 0.0)
                    rows[0, r, pl.ds(j * 16, 16)] = h
                return carry

            lax.fori_loop(0, FIN, fin_u, 0)
            pltpu.sync_copy(bufA, hu.at[pl.ds(c * NPAD + ro, FIN)])

        lax.fori_loop(0, FIN, fill_zero, 0)
        zero_acc(accA)
        plsc.subcore_barrier()

        do_etype(s2c, d2c, tabC, accA)
        plsc.subcore_barrier()

        for k in range(NF):
            ro = rs + k * FIN
            pltpu.sync_copy(accA.at[pl.ds(ro, FIN)], bufA)
            pltpu.sync_copy(recip.at[pl.ds(NPAD + ro, FIN)], rAf)

            def fin_i(r, carry):
                ra = plsc.load_gather(rAf, [jnp.full((16,), r, jnp.int32)])
                for j in range(4):
                    h = rows[0, r, pl.ds(j * 16, 16)] * ra
                    if relu:
                        h = jnp.maximum(h, 0.0)
                    rows[0, r, pl.ds(j * 16, 16)] = h
                return carry

            lax.fori_loop(0, FIN, fin_i, 0)
            pltpu.sync_copy(bufA, hi_.at[pl.ds(c * NPAD + ro, FIN)])

    return pl.kernel(
        body,
        out_type=[jax.ShapeDtypeStruct((2 * NPAD, H), jnp.float32)] * 2,
        mesh=_MESH,
        compiler_params=_SC_PARAMS,
        scratch_types=[
            pltpu.VMEM_SHARED((NPAD, H), jnp.float32),
            pltpu.VMEM_SHARED((NPAD, H), jnp.float32),
            pltpu.VMEM((CPT, CH), jnp.int32),
            pltpu.VMEM((CPT, CH), jnp.int32),
            pltpu.VMEM((2, CH, H), jnp.float32),
            pltpu.VMEM((FIN,), jnp.float32),
            pltpu.VMEM((FIN,), jnp.float32),
            pltpu.SemaphoreType.DMA,
            pltpu.SemaphoreType.DMA,
        ],
    )


_layer_kernel_relu = _make_layer_kernel(True, N)
_layer_kernel_lin = _make_layer_kernel(False, NPAD)

BM = 1000
BM2 = 1024


def _csel(x, h):
    return jnp.where(h == 0, x[:, :H], x[:, H:])


def _tc1_body(xu, xi, wf, wc, wcb, bf, bc, bcb, tf, tc, tcb):
    h = pl.program_id(0) // 10

    def mm(x, w, b):
        return (jnp.dot(x[...], _csel(w[...], h),
                        preferred_element_type=jnp.float32) + _csel(b[...], h))

    tf[...] = mm(xu, wf, bf)
    tc[...] = mm(xu, wc, bc)
    tcb[...] = mm(xi, wcb, bcb)


def _tc2_body(xu_lo, xu_hi, xi_lo, xi_hi, wf, wc, wcb, bf, bc, bcb, tf, tc, tcb):
    h = pl.program_id(0) // 10

    def mm(lo, hi, w, b):
        w64 = _csel(w[...], h)
        return (jnp.dot(lo[...], w64[:H, :], preferred_element_type=jnp.float32)
                + jnp.dot(hi[...], w64[H:, :], preferred_element_type=jnp.float32)
                + _csel(b[...], h))

    tf[...] = mm(xu_lo, xu_hi, wf, bf)
    tc[...] = mm(xu_lo, xu_hi, wc, bc)
    tcb[...] = mm(xi_lo, xi_hi, wcb, bcb)


_x_spec = pl.BlockSpec((BM, D), lambda g: (g % 10, 0))
_w_spec = pl.BlockSpec((D, D), lambda g: (0, 0))
_b_spec = pl.BlockSpec((1, D), lambda g: (0, 0))

_tc1 = pl.pallas_call(
    _tc1_body,
    grid=(20,),
    in_specs=[_x_spec, _x_spec, _w_spec, _w_spec, _w_spec,
              _b_spec, _b_spec, _b_spec],
    out_specs=[pl.BlockSpec((BM, H), lambda g: (g, 0))] * 3,
    out_shape=[jax.ShapeDtypeStruct((2 * N, H), jnp.float32)] * 3,
)

_tc2 = pl.pallas_call(
    _tc2_body,
    grid=(20,),
    in_specs=[pl.BlockSpec((BM2, H), lambda g: (g % 10, 0)),
              pl.BlockSpec((BM2, H), lambda g: (10 + g % 10, 0)),
              pl.BlockSpec((BM2, H), lambda g: (g % 10, 0)),
              pl.BlockSpec((BM2, H), lambda g: (10 + g % 10, 0)),
              _w_spec, _w_spec, _w_spec,
              _b_spec, _b_spec, _b_spec],
    out_specs=[pl.BlockSpec((BM2, H), lambda g: (g, 0))] * 3,
    out_shape=[jax.ShapeDtypeStruct((2 * NPAD, H), jnp.float32)] * 3,
)


def _pad2d(a, pad_val):
    pad = jnp.full((EP - E,), pad_val, jnp.int32)
    return jnp.concatenate([a.astype(jnp.int32), pad]).reshape(NROW, CH)


def kernel(follows_src, follows_dst, clicks_src, clicks_dst,
           clickedby_src, clickedby_dst, emb_user, emb_item,
           W1_follows, b1_follows, W1_clicks, b1_clicks,
           W1_clickedby, b1_clickedby,
           W2_follows, b2_follows, W2_clicks, b2_clicks,
           W2_clickedby, b2_clickedby):
    s2f = _pad2d(follows_src, 0)
    d2f = _pad2d(follows_dst, N)
    s2c = _pad2d(clicks_src, 0)
    d2c = _pad2d(clicks_dst, N)
    s2cb = _pad2d(clickedby_src, 0)
    d2cb = _pad2d(clickedby_dst, N)

    recip = _count_kernel(d2f, d2c, d2cb)

    t1f, t1c, t1cb = _tc1(emb_user, emb_item,
                          W1_follows, W1_clicks, W1_clickedby,
                          b1_follows.reshape(1, D), b1_clicks.reshape(1, D),
                          b1_clickedby.reshape(1, D))
    hu1, hi1 = _layer_kernel_relu(s2f, d2f, s2c, d2c, s2cb, d2cb,
                                  t1f, t1c, t1cb, recip)

    t2f, t2c, t2cb = _tc2(hu1, hu1, hi1, hi1,
                          W2_follows, W2_clicks, W2_clickedby,
                          b2_follows.reshape(1, D), b2_clicks.reshape(1, D),
                          b2_clickedby.reshape(1, D))
    hu2, hi2 = _layer_kernel_lin(s2f, d2f, s2c, d2c, s2cb, d2cb,
                                 t2f, t2c, t2cb, recip)

    h_user = jnp.concatenate([hu2[:N], hu2[NPAD:NPAD + N]], axis=1)
    h_item = jnp.concatenate([hi2[:N], hi2[NPAD:NPAD + N]], axis=1)
    return (h_user, h_item)


# speedup vs baseline: 1.5207x; 1.5207x over previous
import jax
import jax.numpy as jnp
from jax import lax
from jax.experimental import pallas as pl
from jax.experimental.pallas import tpu as pltpu
from jax.experimental.pallas import tpu_sc as plsc

N = 10000
E = 100000
D = 128
H = 64
CH = 64
NSUB = 16
CPT = 100
EP = CH * CPT * NSUB
NROW = EP // CH
NPAD = 10240
TR = NPAD // NSUB
FIN = 64
NF = TR // FIN
CR = 3 * NPAD // NSUB

_MESH = plsc.VectorSubcoreMesh(core_axis_name="c", subcore_axis_name="s")
_SC_PARAMS = pltpu.CompilerParams(use_tc_tiling_on_sc=False, needs_layout_passes=False)


def _count_body(d2_f, d2_c, d2_cb, recip, acc2d, dbuf, cloc, rbuf, tbuf):
    c = lax.axis_index("c")
    s = lax.axis_index("s")

    def zloc(m, carry):
        cloc[pl.ds(m * 16, 16)] = jnp.zeros((16,), jnp.float32)
        return carry

    lax.fori_loop(0, 3 * NPAD // 16, zloc, 0)

    @pl.when(c == 0)
    def _():
        ones16 = jnp.full((16,), 1.0, jnp.float32)
        for e, d2 in enumerate((d2_f, d2_c, d2_cb)):
            pltpu.sync_copy(d2.at[pl.ds(s * CPT, CPT)], dbuf)
            off = e * NPAD

            def hist(r, carry):
                for j in range(CH // 16):
                    v = dbuf[r, pl.ds(j * 16, 16)] + off
                    plsc.addupdate_scatter(cloc, [v], ones16)
                return carry

            lax.fori_loop(0, CPT, hist, 0)
        pltpu.sync_copy(cloc, acc2d.at[s])

    plsc.subcore_barrier()

    @pl.when(c == 0)
    def _():
        def zbuf(m, carry):
            rbuf[pl.ds(m * 16, 16)] = jnp.zeros((16,), jnp.float32)
            return carry

        lax.fori_loop(0, CR // 16, zbuf, 0)
        for t in range(NSUB):
            pltpu.sync_copy(acc2d.at[t, pl.ds(s * CR, CR)], tbuf)

            def accum(m, carry):
                rbuf[pl.ds(m * 16, 16)] = (rbuf[pl.ds(m * 16, 16)]
                                           + tbuf[pl.ds(m * 16, 16)])
                return carry

            lax.fori_loop(0, CR // 16, accum, 0)

        def rec(m, carry):
            v = rbuf[pl.ds(m * 16, 16)]
            rbuf[pl.ds(m * 16, 16)] = 1.0 / jnp.maximum(v, 1.0)
            return carry

        lax.fori_loop(0, CR // 16, rec, 0)
        pltpu.sync_copy(rbuf, recip.at[pl.ds(s * CR, CR)])


_count_kernel = pl.kernel(
    _count_body,
    out_type=jax.ShapeDtypeStruct((3 * NPAD,), jnp.float32),
    mesh=_MESH,
    compiler_params=_SC_PARAMS,
    scratch_types=[
        pltpu.VMEM_SHARED((NSUB, 3 * NPAD), jnp.float32),
        pltpu.VMEM((CPT, CH), jnp.int32),
        pltpu.VMEM((3 * NPAD,), jnp.float32),
        pltpu.VMEM((CR,), jnp.float32),
        pltpu.VMEM((CR,), jnp.float32),
    ],
)


def _make_layer_kernel(relu: bool, tbn: int):
    def body(s2f, d2f, s2c, d2c, s2cb, d2cb,
             tabF, tabC, tabCB, recip,
             hu, hi_,
             accA, accB,
             sbuf, dbuf, rows, bufA, bufB, rAf, rBf, semA, semB):
        c = lax.axis_index("c")
        s = lax.axis_index("s")
        rs = s * TR
        half = c * tbn

        def fill_zero(r, carry):
            for j in range(4):
                bufA[r, pl.ds(j * 16, 16)] = jnp.zeros((16,), jnp.float32)
            return carry

        def zero_acc(acc):
            for k in range(NF):
                pltpu.sync_copy(bufA, acc.at[pl.ds(rs + k * FIN, FIN)])

        lax.fori_loop(0, FIN, fill_zero, 0)
        zero_acc(accA)
        zero_acc(accB)
        plsc.subcore_barrier()

        def do_etype(s2, d2, table, acc):
            pltpu.sync_copy(s2.at[pl.ds(s * CPT, CPT)], sbuf)
            pltpu.sync_copy(d2.at[pl.ds(s * CPT, CPT)], dbuf)

            def addoff(r, carry):
                for j in range(CH // 16):
                    sbuf[r, pl.ds(j * 16, 16)] = sbuf[r, pl.ds(j * 16, 16)] + half
                return carry

            lax.fori_loop(0, CPT, addoff, 0)

            pltpu.async_copy(table.at[sbuf.at[0]], rows.at[0], semA)
            pltpu.async_copy(table.at[sbuf.at[1]], rows.at[1], semB)

            def grp(g, carry):
                for k, sem in ((0, semA), (1, semB)):
                    ch = g * 2 + k
                    pltpu.make_async_copy(
                        table.at[sbuf.at[0]], rows.at[k], sem).wait()
                    pltpu.sync_copy(rows.at[k], acc.at[dbuf.at[ch]], add=True)
                    nxt = ch + 2

                    @pl.when(nxt < CPT)
                    def _():
                        pltpu.async_copy(
                            table.at[sbuf.at[nxt]], rows.at[k], sem)
                return carry

            lax.fori_loop(0, CPT // 2, grp, 0)

        do_etype(s2f, d2f, tabF, accA)
        do_etype(s2cb, d2cb, tabCB, accB)
        plsc.subcore_barrier()

        for k in range(NF):
            ro = rs + k * FIN
            pltpu.sync_copy(accA.at[pl.ds(ro, FIN)], bufA)
            pltpu.sync_copy(accB.at[pl.ds(ro, FIN)], bufB)
            pltpu.sync_copy(recip.at[pl.ds(ro, FIN)], rAf)
            pltpu.sync_copy(recip.at[pl.ds(2 * NPAD + ro, FIN)], rBf)

            def fin_u(r, carry):
                ri = jnp.full((16,), r, jnp.int32)
                ra = plsc.load_gather(rAf, [ri])
                rb = plsc.load_gather(rBf, [ri])
                for j in range(4):
                    a = bufA[r, pl.ds(j * 16, 16)]
                    b = bufB[r, pl.ds(j * 16, 16)]
                    h = 0.5 * (a * ra + b * rb)
                    if relu:
                        h = jnp.maximum(h, 0.0)
                    bufA[r, pl.ds(j * 16, 16)] = h
                return carry

            lax.fori_loop(0, FIN, fin_u, 0)
            pltpu.sync_copy(bufA, hu.at[pl.ds(c * NPAD + ro, FIN)])

        lax.fori_loop(0, FIN, fill_zero, 0)
        zero_acc(accA)
        plsc.subcore_barrier()

        do_etype(s2c, d2c, tabC, accA)
        plsc.subcore_barrier()

        for k in range(NF):
            ro = rs + k * FIN
            pltpu.sync_copy(accA.at[pl.ds(ro, FIN)], bufA)
            pltpu.sync_copy(recip.at[pl.ds(NPAD + ro, FIN)], rAf)

            def fin_i(r, carry):
                ra = plsc.load_gather(rAf, [jnp.full((16,), r, jnp.int32)])
                for j in range(4):
                    h = bufA[r, pl.ds(j * 16, 16)] * ra
                    if relu:
                        h = jnp.maximum(h, 0.0)
                    bufA[r, pl.ds(j * 16, 16)] = h
                return carry

            lax.fori_loop(0, FIN, fin_i, 0)
            pltpu.sync_copy(bufA, hi_.at[pl.ds(c * NPAD + ro, FIN)])

    return pl.kernel(
        body,
        out_type=[jax.ShapeDtypeStruct((2 * NPAD, H), jnp.float32)] * 2,
        mesh=_MESH,
        compiler_params=_SC_PARAMS,
        scratch_types=[
            pltpu.VMEM_SHARED((NPAD, H), jnp.float32),
            pltpu.VMEM_SHARED((NPAD, H), jnp.float32),
            pltpu.VMEM((CPT, CH), jnp.int32),
            pltpu.VMEM((CPT, CH), jnp.int32),
            pltpu.VMEM((2, CH, H), jnp.float32),
            pltpu.VMEM((FIN, H), jnp.float32),
            pltpu.VMEM((FIN, H), jnp.float32),
            pltpu.VMEM((FIN,), jnp.float32),
            pltpu.VMEM((FIN,), jnp.float32),
            pltpu.SemaphoreType.DMA,
            pltpu.SemaphoreType.DMA,
        ],
    )


_layer_kernel_relu = _make_layer_kernel(True, N)
_layer_kernel_lin = _make_layer_kernel(False, NPAD)

BM = 1000
BM2 = 1024


def _csel(x, h):
    return jnp.where(h == 0, x[:, :H], x[:, H:])


def _tc1_body(xu, xi, wf, wc, wcb, bf, bc, bcb, tf, tc, tcb):
    h = pl.program_id(0) // 10

    def mm(x, w, b):
        return (jnp.dot(x[...], _csel(w[...], h),
                        preferred_element_type=jnp.float32) + _csel(b[...], h))

    tf[...] = mm(xu, wf, bf)
    tc[...] = mm(xu, wc, bc)
    tcb[...] = mm(xi, wcb, bcb)


def _tc2_body(xu_lo, xu_hi, xi_lo, xi_hi, wf, wc, wcb, bf, bc, bcb, tf, tc, tcb):
    h = pl.program_id(0) // 10

    def mm(lo, hi, w, b):
        w64 = _csel(w[...], h)
        return (jnp.dot(lo[...], w64[:H, :], preferred_element_type=jnp.float32)
                + jnp.dot(hi[...], w64[H:, :], preferred_element_type=jnp.float32)
                + _csel(b[...], h))

    tf[...] = mm(xu_lo, xu_hi, wf, bf)
    tc[...] = mm(xu_lo, xu_hi, wc, bc)
    tcb[...] = mm(xi_lo, xi_hi, wcb, bcb)


_x_spec = pl.BlockSpec((BM, D), lambda g: (g % 10, 0))
_w_spec = pl.BlockSpec((D, D), lambda g: (0, 0))
_b_spec = pl.BlockSpec((1, D), lambda g: (0, 0))

_tc1 = pl.pallas_call(
    _tc1_body,
    grid=(20,),
    in_specs=[_x_spec, _x_spec, _w_spec, _w_spec, _w_spec,
              _b_spec, _b_spec, _b_spec],
    out_specs=[pl.BlockSpec((BM, H), lambda g: (g, 0))] * 3,
    out_shape=[jax.ShapeDtypeStruct((2 * N, H), jnp.float32)] * 3,
)

_tc2 = pl.pallas_call(
    _tc2_body,
    grid=(20,),
    in_specs=[pl.BlockSpec((BM2, H), lambda g: (g % 10, 0)),
              pl.BlockSpec((BM2, H), lambda g: (10 + g % 10, 0)),
              pl.BlockSpec((BM2, H), lambda g: (g % 10, 0)),
              pl.BlockSpec((BM2, H), lambda g: (10 + g % 10, 0)),
              _w_spec, _w_spec, _w_spec,
              _b_spec, _b_spec, _b_spec],
    out_specs=[pl.BlockSpec((BM2, H), lambda g: (g, 0))] * 3,
    out_shape=[jax.ShapeDtypeStruct((2 * NPAD, H), jnp.float32)] * 3,
)


def _pad2d(a, pad_val):
    pad = jnp.full((EP - E,), pad_val, jnp.int32)
    return jnp.concatenate([a.astype(jnp.int32), pad]).reshape(NROW, CH)


def kernel(follows_src, follows_dst, clicks_src, clicks_dst,
           clickedby_src, clickedby_dst, emb_user, emb_item,
           W1_follows, b1_follows, W1_clicks, b1_clicks,
           W1_clickedby, b1_clickedby,
           W2_follows, b2_follows, W2_clicks, b2_clicks,
           W2_clickedby, b2_clickedby):
    s2f = _pad2d(follows_src, 0)
    d2f = _pad2d(follows_dst, N)
    s2c = _pad2d(clicks_src, 0)
    d2c = _pad2d(clicks_dst, N)
    s2cb = _pad2d(clickedby_src, 0)
    d2cb = _pad2d(clickedby_dst, N)

    recip = _count_kernel(d2f, d2c, d2cb)

    t1f, t1c, t1cb = _tc1(emb_user, emb_item,
                          W1_follows, W1_clicks, W1_clickedby,
                          b1_follows.reshape(1, D), b1_clicks.reshape(1, D),
                          b1_clickedby.reshape(1, D))
    hu1, hi1 = _layer_kernel_relu(s2f, d2f, s2c, d2c, s2cb, d2cb,
                                  t1f, t1c, t1cb, recip)

    t2f, t2c, t2cb = _tc2(hu1, hu1, hi1, hi1,
                          W2_follows, W2_clicks, W2_clickedby,
                          b2_follows.reshape(1, D), b2_clicks.reshape(1, D),
                          b2_clickedby.reshape(1, D))
    hu2, hi2 = _layer_kernel_lin(s2f, d2f, s2c, d2c, s2cb, d2cb,
                                 t2f, t2c, t2cb, recip)

    h_user = jnp.concatenate([hu2[:N], hu2[NPAD:NPAD + N]], axis=1)
    h_item = jnp.concatenate([hi2[:N], hi2[NPAD:NPAD + N]], axis=1)
    return (h_user, h_item)


# speedup vs baseline: 2.2934x; 1.5081x over previous
import jax
import jax.numpy as jnp
from jax import lax
from jax.experimental import pallas as pl
from jax.experimental.pallas import tpu as pltpu
from jax.experimental.pallas import tpu_sc as plsc

N = 10000
E = 100000
D = 128
H = 64
CH = 128
NSUB = 16
CPT = 50
EP = CH * CPT * NSUB
NROW = EP // CH
NPAD = 10240
TR = NPAD // NSUB
FIN = 64
NF = TR // FIN
CR = 3 * NPAD // NSUB

_MESH = plsc.VectorSubcoreMesh(core_axis_name="c", subcore_axis_name="s")
_SC_PARAMS = pltpu.CompilerParams(use_tc_tiling_on_sc=False, needs_layout_passes=False)


def _count_body(d2_f, d2_c, d2_cb, recip, acc2d, dbuf, cloc, rbuf, tbuf):
    c = lax.axis_index("c")
    s = lax.axis_index("s")

    def zloc(m, carry):
        cloc[pl.ds(m * 16, 16)] = jnp.zeros((16,), jnp.float32)
        return carry

    lax.fori_loop(0, 3 * NPAD // 16, zloc, 0)

    @pl.when(c == 0)
    def _():
        ones16 = jnp.full((16,), 1.0, jnp.float32)
        for e, d2 in enumerate((d2_f, d2_c, d2_cb)):
            pltpu.sync_copy(d2.at[pl.ds(s * CPT, CPT)], dbuf)
            off = e * NPAD

            def hist(r, carry):
                for j in range(CH // 16):
                    v = dbuf[r, pl.ds(j * 16, 16)] + off
                    plsc.addupdate_scatter(cloc, [v], ones16)
                return carry

            lax.fori_loop(0, CPT, hist, 0)
        pltpu.sync_copy(cloc, acc2d.at[s])

    plsc.subcore_barrier()

    @pl.when(c == 0)
    def _():
        def zbuf(m, carry):
            rbuf[pl.ds(m * 16, 16)] = jnp.zeros((16,), jnp.float32)
            return carry

        lax.fori_loop(0, CR // 16, zbuf, 0)
        for t in range(NSUB):
            pltpu.sync_copy(acc2d.at[t, pl.ds(s * CR, CR)], tbuf)

            def accum(m, carry):
                rbuf[pl.ds(m * 16, 16)] = (rbuf[pl.ds(m * 16, 16)]
                                           + tbuf[pl.ds(m * 16, 16)])
                return carry

            lax.fori_loop(0, CR // 16, accum, 0)

        def rec(m, carry):
            v = rbuf[pl.ds(m * 16, 16)]
            rbuf[pl.ds(m * 16, 16)] = 1.0 / jnp.maximum(v, 1.0)
            return carry

        lax.fori_loop(0, CR // 16, rec, 0)
        pltpu.sync_copy(rbuf, recip.at[pl.ds(s * CR, CR)])


_count_kernel = pl.kernel(
    _count_body,
    out_type=jax.ShapeDtypeStruct((3 * NPAD,), jnp.float32),
    mesh=_MESH,
    compiler_params=_SC_PARAMS,
    scratch_types=[
        pltpu.VMEM_SHARED((NSUB, 3 * NPAD), jnp.float32),
        pltpu.VMEM((CPT, CH), jnp.int32),
        pltpu.VMEM((3 * NPAD,), jnp.float32),
        pltpu.VMEM((CR,), jnp.float32),
        pltpu.VMEM((CR,), jnp.float32),
    ],
)


def _make_layer_kernel(relu: bool, tbn: int):
    RPT = None

    def body(s2f, d2f, s2c, d2c, s2cb, d2cb,
             tabF, tabC, tabCB, recip,
             hu, hi_,
             tabS, accA,
             sbuf, dbuf, rows, bufA, bufB, rAf, rBf, semA, semB):
        c = lax.axis_index("c")
        s = lax.axis_index("s")
        rs = s * TR
        rpt = tbn // NSUB

        def fill_zero(r, carry):
            for j in range(4):
                bufA[r, pl.ds(j * 16, 16)] = jnp.zeros((16,), jnp.float32)
            return carry

        def zero_acc():
            lax.fori_loop(0, FIN, fill_zero, 0)
            for k in range(NF):
                pltpu.sync_copy(bufA, accA.at[pl.ds(rs + k * FIN, FIN)])

        def load_tab(table):
            pltpu.sync_copy(table.at[pl.ds(c * tbn + s * rpt, rpt)],
                            tabS.at[pl.ds(s * rpt, rpt)])

        def scat(s2, d2):
            pltpu.sync_copy(s2.at[pl.ds(s * CPT, CPT)], sbuf)
            pltpu.sync_copy(d2.at[pl.ds(s * CPT, CPT)], dbuf)

            pltpu.async_copy(tabS.at[sbuf.at[0]], rows.at[0], semA)
            pltpu.async_copy(tabS.at[sbuf.at[1]], rows.at[1], semB)

            def grp(g, carry):
                for k, sem in ((0, semA), (1, semB)):
                    ch = g * 2 + k
                    pltpu.make_async_copy(
                        tabS.at[sbuf.at[0]], rows.at[k], sem).wait()
                    pltpu.sync_copy(rows.at[k], accA.at[dbuf.at[ch]], add=True)
                    nxt = ch + 2

                    @pl.when(nxt < CPT)
                    def _():
                        pltpu.async_copy(
                            tabS.at[sbuf.at[nxt]], rows.at[k], sem)
                return carry

            lax.fori_loop(0, CPT // 2, grp, 0)

        load_tab(tabF)
        zero_acc()
        plsc.subcore_barrier()
        scat(s2f, d2f)
        plsc.subcore_barrier()
        for k in range(NF):
            ro = rs + k * FIN
            pltpu.sync_copy(accA.at[pl.ds(ro, FIN)], bufA)
            pltpu.sync_copy(recip.at[pl.ds(ro, FIN)], rAf)

            def fin_f(r, carry):
                ra = plsc.load_gather(rAf, [jnp.full((16,), r, jnp.int32)])
                for j in range(4):
                    h = 0.5 * (bufA[r, pl.ds(j * 16, 16)] * ra)
                    bufA[r, pl.ds(j * 16, 16)] = h
                return carry

            lax.fori_loop(0, FIN, fin_f, 0)
            pltpu.sync_copy(bufA, hu.at[pl.ds(c * NPAD + ro, FIN)])

        load_tab(tabCB)
        zero_acc()
        plsc.subcore_barrier()
        scat(s2cb, d2cb)
        plsc.subcore_barrier()
        for k in range(NF):
            ro = rs + k * FIN
            pltpu.sync_copy(accA.at[pl.ds(ro, FIN)], bufA)
            pltpu.sync_copy(hu.at[pl.ds(c * NPAD + ro, FIN)], bufB)
            pltpu.sync_copy(recip.at[pl.ds(2 * NPAD + ro, FIN)], rAf)

            def fin_u(r, carry):
                ra = plsc.load_gather(rAf, [jnp.full((16,), r, jnp.int32)])
                for j in range(4):
                    h = (bufB[r, pl.ds(j * 16, 16)]
                         + 0.5 * (bufA[r, pl.ds(j * 16, 16)] * ra))
                    if relu:
                        h = jnp.maximum(h, 0.0)
                    bufA[r, pl.ds(j * 16, 16)] = h
                return carry

            lax.fori_loop(0, FIN, fin_u, 0)
            pltpu.sync_copy(bufA, hu.at[pl.ds(c * NPAD + ro, FIN)])

        load_tab(tabC)
        zero_acc()
        plsc.subcore_barrier()
        scat(s2c, d2c)
        plsc.subcore_barrier()
        for k in range(NF):
            ro = rs + k * FIN
            pltpu.sync_copy(accA.at[pl.ds(ro, FIN)], bufA)
            pltpu.sync_copy(recip.at[pl.ds(NPAD + ro, FIN)], rAf)

            def fin_i(r, carry):
                ra = plsc.load_gather(rAf, [jnp.full((16,), r, jnp.int32)])
                for j in range(4):
                    h = bufA[r, pl.ds(j * 16, 16)] * ra
                    if relu:
                        h = jnp.maximum(h, 0.0)
                    bufA[r, pl.ds(j * 16, 16)] = h
                return carry

            lax.fori_loop(0, FIN, fin_i, 0)
            pltpu.sync_copy(bufA, hi_.at[pl.ds(c * NPAD + ro, FIN)])

    return pl.kernel(
        body,
        out_type=[jax.ShapeDtypeStruct((2 * NPAD, H), jnp.float32)] * 2,
        mesh=_MESH,
        compiler_params=_SC_PARAMS,
        scratch_types=[
            pltpu.VMEM_SHARED((NPAD, H), jnp.float32),
            pltpu.VMEM_SHARED((NPAD, H), jnp.float32),
            pltpu.VMEM((CPT, CH), jnp.int32),
            pltpu.VMEM((CPT, CH), jnp.int32),
            pltpu.VMEM((2, CH, H), jnp.float32),
            pltpu.VMEM((FIN, H), jnp.float32),
            pltpu.VMEM((FIN, H), jnp.float32),
            pltpu.VMEM((FIN,), jnp.float32),
            pltpu.VMEM((FIN,), jnp.float32),
            pltpu.SemaphoreType.DMA,
            pltpu.SemaphoreType.DMA,
        ],
    )


_layer_kernel_relu = _make_layer_kernel(True, N)
_layer_kernel_lin = _make_layer_kernel(False, NPAD)

BM = 1000
BM2 = 1024


def _csel(x, h):
    return jnp.where(h == 0, x[:, :H], x[:, H:])


def _tc1_body(xu, xi, wf, wc, wcb, bf, bc, bcb, tf, tc, tcb):
    h = pl.program_id(0) // 10

    def mm(x, w, b):
        return (jnp.dot(x[...], _csel(w[...], h),
                        preferred_element_type=jnp.float32) + _csel(b[...], h))

    tf[...] = mm(xu, wf, bf)
    tc[...] = mm(xu, wc, bc)
    tcb[...] = mm(xi, wcb, bcb)


def _tc2_body(xu_lo, xu_hi, xi_lo, xi_hi, wf, wc, wcb, bf, bc, bcb, tf, tc, tcb):
    h = pl.program_id(0) // 10

    def mm(lo, hi, w, b):
        w64 = _csel(w[...], h)
        return (jnp.dot(lo[...], w64[:H, :], preferred_element_type=jnp.float32)
                + jnp.dot(hi[...], w64[H:, :], preferred_element_type=jnp.float32)
                + _csel(b[...], h))

    tf[...] = mm(xu_lo, xu_hi, wf, bf)
    tc[...] = mm(xu_lo, xu_hi, wc, bc)
    tcb[...] = mm(xi_lo, xi_hi, wcb, bcb)


_x_spec = pl.BlockSpec((BM, D), lambda g: (g % 10, 0))
_w_spec = pl.BlockSpec((D, D), lambda g: (0, 0))
_b_spec = pl.BlockSpec((1, D), lambda g: (0, 0))

_tc1 = pl.pallas_call(
    _tc1_body,
    grid=(20,),
    in_specs=[_x_spec, _x_spec, _w_spec, _w_spec, _w_spec,
              _b_spec, _b_spec, _b_spec],
    out_specs=[pl.BlockSpec((BM, H), lambda g: (g, 0))] * 3,
    out_shape=[jax.ShapeDtypeStruct((2 * N, H), jnp.float32)] * 3,
)

_tc2 = pl.pallas_call(
    _tc2_body,
    grid=(20,),
    in_specs=[pl.BlockSpec((BM2, H), lambda g: (g % 10, 0)),
              pl.BlockSpec((BM2, H), lambda g: (10 + g % 10, 0)),
              pl.BlockSpec((BM2, H), lambda g: (g % 10, 0)),
              pl.BlockSpec((BM2, H), lambda g: (10 + g % 10, 0)),
              _w_spec, _w_spec, _w_spec,
              _b_spec, _b_spec, _b_spec],
    out_specs=[pl.BlockSpec((BM2, H), lambda g: (g, 0))] * 3,
    out_shape=[jax.ShapeDtypeStruct((2 * NPAD, H), jnp.float32)] * 3,
)


def _pad2d(a, pad_val):
    pad = jnp.full((EP - E,), pad_val, jnp.int32)
    return jnp.concatenate([a.astype(jnp.int32), pad]).reshape(NROW, CH)


def kernel(follows_src, follows_dst, clicks_src, clicks_dst,
           clickedby_src, clickedby_dst, emb_user, emb_item,
           W1_follows, b1_follows, W1_clicks, b1_clicks,
           W1_clickedby, b1_clickedby,
           W2_follows, b2_follows, W2_clicks, b2_clicks,
           W2_clickedby, b2_clickedby):
    s2f = _pad2d(follows_src, 0)
    d2f = _pad2d(follows_dst, N)
    s2c = _pad2d(clicks_src, 0)
    d2c = _pad2d(clicks_dst, N)
    s2cb = _pad2d(clickedby_src, 0)
    d2cb = _pad2d(clickedby_dst, N)

    recip = _count_kernel(d2f, d2c, d2cb)

    t1f, t1c, t1cb = _tc1(emb_user, emb_item,
                          W1_follows, W1_clicks, W1_clickedby,
                          b1_follows.reshape(1, D), b1_clicks.reshape(1, D),
                          b1_clickedby.reshape(1, D))
    hu1, hi1 = _layer_kernel_relu(s2f, d2f, s2c, d2c, s2cb, d2cb,
                                  t1f, t1c, t1cb, recip)

    t2f, t2c, t2cb = _tc2(hu1, hu1, hi1, hi1,
                          W2_follows, W2_clicks, W2_clickedby,
                          b2_follows.reshape(1, D), b2_clicks.reshape(1, D),
                          b2_clickedby.reshape(1, D))
    hu2, hi2 = _layer_kernel_lin(s2f, d2f, s2c, d2c, s2cb, d2cb,
                                 t2f, t2c, t2cb, recip)

    h_user = jnp.concatenate([hu2[:N], hu2[NPAD:NPAD + N]], axis=1)
    h_item = jnp.concatenate([hi2[:N], hi2[NPAD:NPAD + N]], axis=1)
    return (h_user, h_item)


# speedup vs baseline: 2.3208x; 1.0120x over previous
import jax
import jax.numpy as jnp
from jax import lax
from jax.experimental import pallas as pl
from jax.experimental.pallas import tpu as pltpu
from jax.experimental.pallas import tpu_sc as plsc

N = 10000
E = 100000
D = 128
H = 64
CH = 128
NSUB = 16
CPT = 51
EP = CH * CPT * NSUB
NROW = EP // CH
NPAD = 10240
TR = NPAD // NSUB
FIN = 64
NF = TR // FIN
CR = 3 * NPAD // NSUB

_MESH = plsc.VectorSubcoreMesh(core_axis_name="c", subcore_axis_name="s")
_SC_PARAMS = pltpu.CompilerParams(use_tc_tiling_on_sc=False, needs_layout_passes=False)


def _count_body(d2_f, d2_c, d2_cb, recip, acc2d, dbuf, cloc, rbuf, tbuf):
    c = lax.axis_index("c")
    s = lax.axis_index("s")

    def zloc(m, carry):
        cloc[pl.ds(m * 16, 16)] = jnp.zeros((16,), jnp.float32)
        return carry

    lax.fori_loop(0, 3 * NPAD // 16, zloc, 0)

    @pl.when(c == 0)
    def _():
        ones16 = jnp.full((16,), 1.0, jnp.float32)
        for e, d2 in enumerate((d2_f, d2_c, d2_cb)):
            pltpu.sync_copy(d2.at[pl.ds(s * CPT, CPT)], dbuf)
            off = e * NPAD

            def hist(r, carry):
                for j in range(CH // 16):
                    v = dbuf[r, pl.ds(j * 16, 16)] + off
                    plsc.addupdate_scatter(cloc, [v], ones16)
                return carry

            lax.fori_loop(0, CPT, hist, 0)
        pltpu.sync_copy(cloc, acc2d.at[s])

    plsc.subcore_barrier()

    @pl.when(c == 0)
    def _():
        def zbuf(m, carry):
            rbuf[pl.ds(m * 16, 16)] = jnp.zeros((16,), jnp.float32)
            return carry

        lax.fori_loop(0, CR // 16, zbuf, 0)
        for t in range(NSUB):
            pltpu.sync_copy(acc2d.at[t, pl.ds(s * CR, CR)], tbuf)

            def accum(m, carry):
                rbuf[pl.ds(m * 16, 16)] = (rbuf[pl.ds(m * 16, 16)]
                                           + tbuf[pl.ds(m * 16, 16)])
                return carry

            lax.fori_loop(0, CR // 16, accum, 0)

        def rec(m, carry):
            v = rbuf[pl.ds(m * 16, 16)]
            rbuf[pl.ds(m * 16, 16)] = 1.0 / jnp.maximum(v, 1.0)
            return carry

        lax.fori_loop(0, CR // 16, rec, 0)
        pltpu.sync_copy(rbuf, recip.at[pl.ds(s * CR, CR)])


_count_kernel = pl.kernel(
    _count_body,
    out_type=jax.ShapeDtypeStruct((3 * NPAD,), jnp.float32),
    mesh=_MESH,
    compiler_params=_SC_PARAMS,
    scratch_types=[
        pltpu.VMEM_SHARED((NSUB, 3 * NPAD), jnp.float32),
        pltpu.VMEM((CPT, CH), jnp.int32),
        pltpu.VMEM((3 * NPAD,), jnp.float32),
        pltpu.VMEM((CR,), jnp.float32),
        pltpu.VMEM((CR,), jnp.float32),
    ],
)


def _make_layer_kernel(relu: bool, tbn: int):
    RPT = None

    def body(s2f, d2f, s2c, d2c, s2cb, d2cb,
             tabF, tabC, tabCB, recip,
             hu, hi_,
             tabS, accA,
             sbuf, dbuf, rows, bufA, bufB, rAf, rBf, semA, semB, semC):
        c = lax.axis_index("c")
        s = lax.axis_index("s")
        rs = s * TR
        rpt = tbn // NSUB

        def fill_zero(r, carry):
            for j in range(4):
                bufA[r, pl.ds(j * 16, 16)] = jnp.zeros((16,), jnp.float32)
            return carry

        def zero_acc():
            lax.fori_loop(0, FIN, fill_zero, 0)
            for k in range(NF):
                pltpu.sync_copy(bufA, accA.at[pl.ds(rs + k * FIN, FIN)])

        def load_tab(table):
            pltpu.sync_copy(table.at[pl.ds(c * tbn + s * rpt, rpt)],
                            tabS.at[pl.ds(s * rpt, rpt)])

        def scat(s2, d2):
            pltpu.sync_copy(s2.at[pl.ds(s * CPT, CPT)], sbuf)
            pltpu.sync_copy(d2.at[pl.ds(s * CPT, CPT)], dbuf)

            pltpu.async_copy(tabS.at[sbuf.at[0]], rows.at[0], semA)
            pltpu.async_copy(tabS.at[sbuf.at[1]], rows.at[1], semB)
            pltpu.async_copy(tabS.at[sbuf.at[2]], rows.at[2], semC)

            def grp(g, carry):
                for k, sem in ((0, semA), (1, semB), (2, semC)):
                    ch = g * 3 + k
                    pltpu.make_async_copy(
                        tabS.at[sbuf.at[0]], rows.at[k], sem).wait()
                    pltpu.sync_copy(rows.at[k], accA.at[dbuf.at[ch]], add=True)
                    nxt = ch + 3

                    @pl.when(nxt < CPT)
                    def _():
                        pltpu.async_copy(
                            tabS.at[sbuf.at[nxt]], rows.at[k], sem)
                return carry

            lax.fori_loop(0, CPT // 3, grp, 0)

        load_tab(tabF)
        zero_acc()
        plsc.subcore_barrier()
        scat(s2f, d2f)
        plsc.subcore_barrier()
        for k in range(NF):
            ro = rs + k * FIN
            pltpu.sync_copy(accA.at[pl.ds(ro, FIN)], bufA)
            pltpu.sync_copy(recip.at[pl.ds(ro, FIN)], rAf)

            def fin_f(r, carry):
                ra = plsc.load_gather(rAf, [jnp.full((16,), r, jnp.int32)])
                for j in range(4):
                    h = 0.5 * (bufA[r, pl.ds(j * 16, 16)] * ra)
                    bufA[r, pl.ds(j * 16, 16)] = h
                return carry

            lax.fori_loop(0, FIN, fin_f, 0)
            pltpu.sync_copy(bufA, hu.at[pl.ds(c * NPAD + ro, FIN)])

        load_tab(tabCB)
        zero_acc()
        plsc.subcore_barrier()
        scat(s2cb, d2cb)
        plsc.subcore_barrier()
        for k in range(NF):
            ro = rs + k * FIN
            pltpu.sync_copy(accA.at[pl.ds(ro, FIN)], bufA)
            pltpu.sync_copy(hu.at[pl.ds(c * NPAD + ro, FIN)], bufB)
            pltpu.sync_copy(recip.at[pl.ds(2 * NPAD + ro, FIN)], rAf)

            def fin_u(r, carry):
                ra = plsc.load_gather(rAf, [jnp.full((16,), r, jnp.int32)])
                for j in range(4):
                    h = (bufB[r, pl.ds(j * 16, 16)]
                         + 0.5 * (bufA[r, pl.ds(j * 16, 16)] * ra))
                    if relu:
                        h = jnp.maximum(h, 0.0)
                    bufA[r, pl.ds(j * 16, 16)] = h
                return carry

            lax.fori_loop(0, FIN, fin_u, 0)
            pltpu.sync_copy(bufA, hu.at[pl.ds(c * NPAD + ro, FIN)])

        load_tab(tabC)
        zero_acc()
        plsc.subcore_barrier()
        scat(s2c, d2c)
        plsc.subcore_barrier()
        for k in range(NF):
            ro = rs + k * FIN
            pltpu.sync_copy(accA.at[pl.ds(ro, FIN)], bufA)
            pltpu.sync_copy(recip.at[pl.ds(NPAD + ro, FIN)], rAf)

            def fin_i(r, carry):
                ra = plsc.load_gather(rAf, [jnp.full((16,), r, jnp.int32)])
                for j in range(4):
                    h = bufA[r, pl.ds(j * 16, 16)] * ra
                    if relu:
                        h = jnp.maximum(h, 0.0)
                    bufA[r, pl.ds(j * 16, 16)] = h
                return carry

            lax.fori_loop(0, FIN, fin_i, 0)
            pltpu.sync_copy(bufA, hi_.at[pl.ds(c * NPAD + ro, FIN)])

    return pl.kernel(
        body,
        out_type=[jax.ShapeDtypeStruct((2 * NPAD, H), jnp.float32)] * 2,
        mesh=_MESH,
        compiler_params=_SC_PARAMS,
        scratch_types=[
            pltpu.VMEM_SHARED((NPAD, H), jnp.float32),
            pltpu.VMEM_SHARED((NPAD, H), jnp.float32),
            pltpu.VMEM((CPT, CH), jnp.int32),
            pltpu.VMEM((CPT, CH), jnp.int32),
            pltpu.VMEM((3, CH, H), jnp.float32),
            pltpu.VMEM((FIN, H), jnp.float32),
            pltpu.VMEM((FIN, H), jnp.float32),
            pltpu.VMEM((FIN,), jnp.float32),
            pltpu.VMEM((FIN,), jnp.float32),
            pltpu.SemaphoreType.DMA,
            pltpu.SemaphoreType.DMA,
            pltpu.SemaphoreType.DMA,
        ],
    )


_layer_kernel_relu = _make_layer_kernel(True, N)
_layer_kernel_lin = _make_layer_kernel(False, NPAD)

BM = 1000
BM2 = 1024


def _csel(x, h):
    return jnp.where(h == 0, x[:, :H], x[:, H:])


def _tc1_body(xu, xi, wf, wc, wcb, bf, bc, bcb, tf, tc, tcb):
    h = pl.program_id(0) // 10

    def mm(x, w, b):
        return (jnp.dot(x[...], _csel(w[...], h),
                        preferred_element_type=jnp.float32) + _csel(b[...], h))

    tf[...] = mm(xu, wf, bf)
    tc[...] = mm(xu, wc, bc)
    tcb[...] = mm(xi, wcb, bcb)


def _tc2_body(xu_lo, xu_hi, xi_lo, xi_hi, wf, wc, wcb, bf, bc, bcb, tf, tc, tcb):
    h = pl.program_id(0) // 10

    def mm(lo, hi, w, b):
        w64 = _csel(w[...], h)
        return (jnp.dot(lo[...], w64[:H, :], preferred_element_type=jnp.float32)
                + jnp.dot(hi[...], w64[H:, :], preferred_element_type=jnp.float32)
                + _csel(b[...], h))

    tf[...] = mm(xu_lo, xu_hi, wf, bf)
    tc[...] = mm(xu_lo, xu_hi, wc, bc)
    tcb[...] = mm(xi_lo, xi_hi, wcb, bcb)


_x_spec = pl.BlockSpec((BM, D), lambda g: (g % 10, 0))
_w_spec = pl.BlockSpec((D, D), lambda g: (0, 0))
_b_spec = pl.BlockSpec((1, D), lambda g: (0, 0))

_tc1 = pl.pallas_call(
    _tc1_body,
    grid=(20,),
    in_specs=[_x_spec, _x_spec, _w_spec, _w_spec, _w_spec,
              _b_spec, _b_spec, _b_spec],
    out_specs=[pl.BlockSpec((BM, H), lambda g: (g, 0))] * 3,
    out_shape=[jax.ShapeDtypeStruct((2 * N, H), jnp.float32)] * 3,
)

_tc2 = pl.pallas_call(
    _tc2_body,
    grid=(20,),
    in_specs=[pl.BlockSpec((BM2, H), lambda g: (g % 10, 0)),
              pl.BlockSpec((BM2, H), lambda g: (10 + g % 10, 0)),
              pl.BlockSpec((BM2, H), lambda g: (g % 10, 0)),
              pl.BlockSpec((BM2, H), lambda g: (10 + g % 10, 0)),
              _w_spec, _w_spec, _w_spec,
              _b_spec, _b_spec, _b_spec],
    out_specs=[pl.BlockSpec((BM2, H), lambda g: (g, 0))] * 3,
    out_shape=[jax.ShapeDtypeStruct((2 * NPAD, H), jnp.float32)] * 3,
)


def _pad2d(a, pad_val):
    pad = jnp.full((EP - E,), pad_val, jnp.int32)
    return jnp.concatenate([a.astype(jnp.int32), pad]).reshape(NROW, CH)


def kernel(follows_src, follows_dst, clicks_src, clicks_dst,
           clickedby_src, clickedby_dst, emb_user, emb_item,
           W1_follows, b1_follows, W1_clicks, b1_clicks,
           W1_clickedby, b1_clickedby,
           W2_follows, b2_follows, W2_clicks, b2_clicks,
           W2_clickedby, b2_clickedby):
    s2f = _pad2d(follows_src, 0)
    d2f = _pad2d(follows_dst, N)
    s2c = _pad2d(clicks_src, 0)
    d2c = _pad2d(clicks_dst, N)
    s2cb = _pad2d(clickedby_src, 0)
    d2cb = _pad2d(clickedby_dst, N)

    recip = _count_kernel(d2f, d2c, d2cb)

    t1f, t1c, t1cb = _tc1(emb_user, emb_item,
                          W1_follows, W1_clicks, W1_clickedby,
                          b1_follows.reshape(1, D), b1_clicks.reshape(1, D),
                          b1_clickedby.reshape(1, D))
    hu1, hi1 = _layer_kernel_relu(s2f, d2f, s2c, d2c, s2cb, d2cb,
                                  t1f, t1c, t1cb, recip)

    t2f, t2c, t2cb = _tc2(hu1, hu1, hi1, hi1,
                          W2_follows, W2_clicks, W2_clickedby,
                          b2_follows.reshape(1, D), b2_clicks.reshape(1, D),
                          b2_clickedby.reshape(1, D))
    hu2, hi2 = _layer_kernel_lin(s2f, d2f, s2c, d2c, s2cb, d2cb,
                                 t2f, t2c, t2cb, recip)

    h_user = jnp.concatenate([hu2[:N], hu2[NPAD:NPAD + N]], axis=1)
    h_item = jnp.concatenate([hi2[:N], hi2[NPAD:NPAD + N]], axis=1)
    return (h_user, h_item)


# speedup vs baseline: 2.4662x; 1.0626x over previous
import jax
import jax.numpy as jnp
from jax import lax
from jax.experimental import pallas as pl
from jax.experimental.pallas import tpu as pltpu
from jax.experimental.pallas import tpu_sc as plsc

N = 10000
E = 100000
D = 128
H = 64
CH = 128
NSUB = 16
CPT = 51
EP = CH * CPT * NSUB
NROW = EP // CH
NPAD = 10240
TR = NPAD // NSUB
FIN = 64
NF = TR // FIN
CR = 3 * NPAD // NSUB

_MESH = plsc.VectorSubcoreMesh(core_axis_name="c", subcore_axis_name="s")
_SC_PARAMS = pltpu.CompilerParams(use_tc_tiling_on_sc=False, needs_layout_passes=False)


def _count_body(d2_f, d2_c, d2_cb, recip, acc2d, dbuf, cloc, rbuf, tbuf):
    c = lax.axis_index("c")
    s = lax.axis_index("s")

    def zloc(m, carry):
        cloc[pl.ds(m * 16, 16)] = jnp.zeros((16,), jnp.float32)
        return carry

    lax.fori_loop(0, 3 * NPAD // 16, zloc, 0)

    @pl.when(c == 0)
    def _():
        ones16 = jnp.full((16,), 1.0, jnp.float32)
        for e, d2 in enumerate((d2_f, d2_c, d2_cb)):
            pltpu.sync_copy(d2.at[pl.ds(s * CPT, CPT)], dbuf)
            off = e * NPAD

            def hist(r, carry):
                for j in range(CH // 16):
                    v = dbuf[r, pl.ds(j * 16, 16)] + off
                    plsc.addupdate_scatter(cloc, [v], ones16)
                return carry

            lax.fori_loop(0, CPT, hist, 0)
        pltpu.sync_copy(cloc, acc2d.at[s])

    plsc.subcore_barrier()

    @pl.when(c == 0)
    def _():
        def zbuf(m, carry):
            rbuf[pl.ds(m * 16, 16)] = jnp.zeros((16,), jnp.float32)
            return carry

        lax.fori_loop(0, CR // 16, zbuf, 0)
        for t in range(NSUB):
            pltpu.sync_copy(acc2d.at[t, pl.ds(s * CR, CR)], tbuf)

            def accum(m, carry):
                rbuf[pl.ds(m * 16, 16)] = (rbuf[pl.ds(m * 16, 16)]
                                           + tbuf[pl.ds(m * 16, 16)])
                return carry

            lax.fori_loop(0, CR // 16, accum, 0)

        def rec(m, carry):
            v = rbuf[pl.ds(m * 16, 16)]
            rbuf[pl.ds(m * 16, 16)] = 1.0 / jnp.maximum(v, 1.0)
            return carry

        lax.fori_loop(0, CR // 16, rec, 0)
        pltpu.sync_copy(rbuf, recip.at[pl.ds(s * CR, CR)])


_count_kernel = pl.kernel(
    _count_body,
    out_type=jax.ShapeDtypeStruct((3 * NPAD,), jnp.float32),
    mesh=_MESH,
    compiler_params=_SC_PARAMS,
    scratch_types=[
        pltpu.VMEM_SHARED((NSUB, 3 * NPAD), jnp.float32),
        pltpu.VMEM((CPT, CH), jnp.int32),
        pltpu.VMEM((3 * NPAD,), jnp.float32),
        pltpu.VMEM((CR,), jnp.float32),
        pltpu.VMEM((CR,), jnp.float32),
    ],
)


def _make_layer_kernel(relu: bool, tbn: int, packed: bool = False):
    RPT = None

    def body(s2f, d2f, s2c, d2c, s2cb, d2cb,
             tabF, tabC, tabCB, recip,
             hu, hi_,
             tabS, accA,
             sbuf, dbuf, rows, bufA, bufB, rAf, rBf, semA, semB, semC):
        c = lax.axis_index("c")
        s = lax.axis_index("s")
        rs = s * TR
        rpt = tbn // NSUB

        def _out_write(out, ro):
            if packed:
                @pl.when(ro + FIN <= N)
                def _():
                    pltpu.sync_copy(
                        bufA, out.at[pl.ds(ro, FIN), pl.ds(c * H, H)])

                @pl.when(ro == (N // FIN) * FIN)
                def _():
                    pltpu.sync_copy(
                        bufA.at[pl.ds(0, N % FIN)],
                        out.at[pl.ds(ro, N % FIN), pl.ds(c * H, H)])
            else:
                pltpu.sync_copy(bufA, out.at[pl.ds(c * NPAD + ro, FIN)])

        def _out_read(out, ro):
            if packed:
                @pl.when(ro + FIN <= N)
                def _():
                    pltpu.sync_copy(
                        out.at[pl.ds(ro, FIN), pl.ds(c * H, H)], bufB)

                @pl.when(ro == (N // FIN) * FIN)
                def _():
                    pltpu.sync_copy(
                        out.at[pl.ds(ro, N % FIN), pl.ds(c * H, H)],
                        bufB.at[pl.ds(0, N % FIN)])
            else:
                pltpu.sync_copy(out.at[pl.ds(c * NPAD + ro, FIN)], bufB)

        def fill_zero(r, carry):
            for j in range(4):
                bufA[r, pl.ds(j * 16, 16)] = jnp.zeros((16,), jnp.float32)
            return carry

        def zero_acc():
            lax.fori_loop(0, FIN, fill_zero, 0)
            for k in range(NF):
                pltpu.sync_copy(bufA, accA.at[pl.ds(rs + k * FIN, FIN)])

        def load_tab(table):
            pltpu.sync_copy(table.at[pl.ds(c * tbn + s * rpt, rpt)],
                            tabS.at[pl.ds(s * rpt, rpt)])

        def scat(s2, d2):
            pltpu.sync_copy(s2.at[pl.ds(s * CPT, CPT)], sbuf)
            pltpu.sync_copy(d2.at[pl.ds(s * CPT, CPT)], dbuf)

            pltpu.async_copy(tabS.at[sbuf.at[0]], rows.at[0], semA)
            pltpu.async_copy(tabS.at[sbuf.at[1]], rows.at[1], semB)
            pltpu.async_copy(tabS.at[sbuf.at[2]], rows.at[2], semC)

            def grp(g, carry):
                for k, sem in ((0, semA), (1, semB), (2, semC)):
                    ch = g * 3 + k
                    pltpu.make_async_copy(
                        tabS.at[sbuf.at[0]], rows.at[k], sem).wait()
                    pltpu.sync_copy(rows.at[k], accA.at[dbuf.at[ch]], add=True)
                    nxt = ch + 3

                    @pl.when(nxt < CPT)
                    def _():
                        pltpu.async_copy(
                            tabS.at[sbuf.at[nxt]], rows.at[k], sem)
                return carry

            lax.fori_loop(0, CPT // 3, grp, 0)

        load_tab(tabF)
        zero_acc()
        plsc.subcore_barrier()
        scat(s2f, d2f)
        plsc.subcore_barrier()
        for k in range(NF):
            ro = rs + k * FIN
            pltpu.sync_copy(accA.at[pl.ds(ro, FIN)], bufA)
            pltpu.sync_copy(recip.at[pl.ds(ro, FIN)], rAf)

            def fin_f(r, carry):
                ra = plsc.load_gather(rAf, [jnp.full((16,), r, jnp.int32)])
                for j in range(4):
                    h = 0.5 * (bufA[r, pl.ds(j * 16, 16)] * ra)
                    bufA[r, pl.ds(j * 16, 16)] = h
                return carry

            lax.fori_loop(0, FIN, fin_f, 0)
            _out_write(hu, ro)

        load_tab(tabCB)
        zero_acc()
        plsc.subcore_barrier()
        scat(s2cb, d2cb)
        plsc.subcore_barrier()
        for k in range(NF):
            ro = rs + k * FIN
            pltpu.sync_copy(accA.at[pl.ds(ro, FIN)], bufA)
            _out_read(hu, ro)
            pltpu.sync_copy(recip.at[pl.ds(2 * NPAD + ro, FIN)], rAf)

            def fin_u(r, carry):
                ra = plsc.load_gather(rAf, [jnp.full((16,), r, jnp.int32)])
                for j in range(4):
                    h = (bufB[r, pl.ds(j * 16, 16)]
                         + 0.5 * (bufA[r, pl.ds(j * 16, 16)] * ra))
                    if relu:
                        h = jnp.maximum(h, 0.0)
                    bufA[r, pl.ds(j * 16, 16)] = h
                return carry

            lax.fori_loop(0, FIN, fin_u, 0)
            _out_write(hu, ro)

        load_tab(tabC)
        zero_acc()
        plsc.subcore_barrier()
        scat(s2c, d2c)
        plsc.subcore_barrier()
        for k in range(NF):
            ro = rs + k * FIN
            pltpu.sync_copy(accA.at[pl.ds(ro, FIN)], bufA)
            pltpu.sync_copy(recip.at[pl.ds(NPAD + ro, FIN)], rAf)

            def fin_i(r, carry):
                ra = plsc.load_gather(rAf, [jnp.full((16,), r, jnp.int32)])
                for j in range(4):
                    h = bufA[r, pl.ds(j * 16, 16)] * ra
                    if relu:
                        h = jnp.maximum(h, 0.0)
                    bufA[r, pl.ds(j * 16, 16)] = h
                return carry

            lax.fori_loop(0, FIN, fin_i, 0)
            _out_write(hi_, ro)

    return pl.kernel(
        body,
        out_type=[jax.ShapeDtypeStruct(
            (N, D) if packed else (2 * NPAD, H), jnp.float32)] * 2,
        mesh=_MESH,
        compiler_params=_SC_PARAMS,
        scratch_types=[
            pltpu.VMEM_SHARED((NPAD, H), jnp.float32),
            pltpu.VMEM_SHARED((NPAD, H), jnp.float32),
            pltpu.VMEM((CPT, CH), jnp.int32),
            pltpu.VMEM((CPT, CH), jnp.int32),
            pltpu.VMEM((3, CH, H), jnp.float32),
            pltpu.VMEM((FIN, H), jnp.float32),
            pltpu.VMEM((FIN, H), jnp.float32),
            pltpu.VMEM((FIN,), jnp.float32),
            pltpu.VMEM((FIN,), jnp.float32),
            pltpu.SemaphoreType.DMA,
            pltpu.SemaphoreType.DMA,
            pltpu.SemaphoreType.DMA,
        ],
    )


_layer_kernel_relu = _make_layer_kernel(True, N)
_layer_kernel_lin = _make_layer_kernel(False, NPAD, packed=True)

BM = 1000
BM2 = 1024


def _csel(x, h):
    return jnp.where(h == 0, x[:, :H], x[:, H:])


def _tc1_body(xu, xi, wf, wc, wcb, bf, bc, bcb, tf, tc, tcb):
    h = pl.program_id(0) // 10

    def mm(x, w, b):
        return (jnp.dot(x[...], _csel(w[...], h),
                        preferred_element_type=jnp.float32) + _csel(b[...], h))

    tf[...] = mm(xu, wf, bf)
    tc[...] = mm(xu, wc, bc)
    tcb[...] = mm(xi, wcb, bcb)


def _tc2_body(xu_lo, xu_hi, xi_lo, xi_hi, wf, wc, wcb, bf, bc, bcb, tf, tc, tcb):
    h = pl.program_id(0) // 10

    def mm(lo, hi, w, b):
        w64 = _csel(w[...], h)
        return (jnp.dot(lo[...], w64[:H, :], preferred_element_type=jnp.float32)
                + jnp.dot(hi[...], w64[H:, :], preferred_element_type=jnp.float32)
                + _csel(b[...], h))

    tf[...] = mm(xu_lo, xu_hi, wf, bf)
    tc[...] = mm(xu_lo, xu_hi, wc, bc)
    tcb[...] = mm(xi_lo, xi_hi, wcb, bcb)


_x_spec = pl.BlockSpec((BM, D), lambda g: (g % 10, 0))
_w_spec = pl.BlockSpec((D, D), lambda g: (0, 0))
_b_spec = pl.BlockSpec((1, D), lambda g: (0, 0))

_tc1 = pl.pallas_call(
    _tc1_body,
    grid=(20,),
    in_specs=[_x_spec, _x_spec, _w_spec, _w_spec, _w_spec,
              _b_spec, _b_spec, _b_spec],
    out_specs=[pl.BlockSpec((BM, H), lambda g: (g, 0))] * 3,
    out_shape=[jax.ShapeDtypeStruct((2 * N, H), jnp.float32)] * 3,
)

_tc2 = pl.pallas_call(
    _tc2_body,
    grid=(20,),
    in_specs=[pl.BlockSpec((BM2, H), lambda g: (g % 10, 0)),
              pl.BlockSpec((BM2, H), lambda g: (10 + g % 10, 0)),
              pl.BlockSpec((BM2, H), lambda g: (g % 10, 0)),
              pl.BlockSpec((BM2, H), lambda g: (10 + g % 10, 0)),
              _w_spec, _w_spec, _w_spec,
              _b_spec, _b_spec, _b_spec],
    out_specs=[pl.BlockSpec((BM2, H), lambda g: (g, 0))] * 3,
    out_shape=[jax.ShapeDtypeStruct((2 * NPAD, H), jnp.float32)] * 3,
)


def _pad2d(a, pad_val):
    pad = jnp.full((EP - E,), pad_val, jnp.int32)
    return jnp.concatenate([a.astype(jnp.int32), pad]).reshape(NROW, CH)


def kernel(follows_src, follows_dst, clicks_src, clicks_dst,
           clickedby_src, clickedby_dst, emb_user, emb_item,
           W1_follows, b1_follows, W1_clicks, b1_clicks,
           W1_clickedby, b1_clickedby,
           W2_follows, b2_follows, W2_clicks, b2_clicks,
           W2_clickedby, b2_clickedby):
    s2f = _pad2d(follows_src, 0)
    d2f = _pad2d(follows_dst, N)
    s2c = _pad2d(clicks_src, 0)
    d2c = _pad2d(clicks_dst, N)
    s2cb = _pad2d(clickedby_src, 0)
    d2cb = _pad2d(clickedby_dst, N)

    recip = _count_kernel(d2f, d2c, d2cb)

    t1f, t1c, t1cb = _tc1(emb_user, emb_item,
                          W1_follows, W1_clicks, W1_clickedby,
                          b1_follows.reshape(1, D), b1_clicks.reshape(1, D),
                          b1_clickedby.reshape(1, D))
    hu1, hi1 = _layer_kernel_relu(s2f, d2f, s2c, d2c, s2cb, d2cb,
                                  t1f, t1c, t1cb, recip)

    t2f, t2c, t2cb = _tc2(hu1, hu1, hi1, hi1,
                          W2_follows, W2_clicks, W2_clickedby,
                          b2_follows.reshape(1, D), b2_clicks.reshape(1, D),
                          b2_clickedby.reshape(1, D))
    hu2, hi2 = _layer_kernel_lin(s2f, d2f, s2c, d2c, s2cb, d2cb,
                                 t2f, t2c, t2cb, recip)

    return (hu2, hi2)
